# baseline (speedup 1.0000x reference)
import jax
import jax.numpy as jnp
from jax.experimental import pallas as pl

_EPS = 1e-8
_NBLK = 32
_BLK = 3125


def _cosine_block(e_ref, x_ref, o_ref):
    x = x_ref[0]
    e = e_ref[0]
    num = jnp.sum(x * e[None, :], axis=1)
    n2 = jnp.sqrt(jnp.sum(x * x, axis=1))
    n1 = jnp.sqrt(jnp.sum(e * e))
    denom = jnp.maximum(n1, _EPS) * jnp.maximum(n2, _EPS)
    o_ref[0, 0, :] = num / denom


def kernel(embedding, x):
    xb = x.reshape(_NBLK, _BLK, 256)
    out = pl.pallas_call(
        _cosine_block,
        grid=(_NBLK,),
        in_specs=[
            pl.BlockSpec((1, 256), lambda i: (0, 0)),
            pl.BlockSpec((1, _BLK, 256), lambda i: (i, 0, 0)),
        ],
        out_specs=pl.BlockSpec((1, 1, _BLK), lambda i: (i, 0, 0)),
        out_shape=jax.ShapeDtypeStruct((_NBLK, 1, _BLK), jnp.float32),
    )(embedding, xb)
    return out.reshape(100000)


# speedup vs baseline: 1.2800x; 1.2800x over previous
import jax
import jax.numpy as jnp
from jax.experimental import pallas as pl

_EPS = 1e-8
_NBLK = 32
_BLK = 3125


def _cosine_block(e_ref, x_ref, o_ref):
    x = x_ref[0]
    e = e_ref[:]
    dims = (((1,), (1,)), ((), ()))
    num = jax.lax.dot_general(e, x, dims,
                              preferred_element_type=jnp.float32)
    ones = jnp.ones((1, 256), jnp.float32)
    n2sq = jax.lax.dot_general(ones, x * x, dims,
                               preferred_element_type=jnp.float32)
    n2 = jnp.sqrt(n2sq)
    n1 = jnp.sqrt(jnp.sum(e * e))
    denom = jnp.maximum(n1, _EPS) * jnp.maximum(n2, _EPS)
    o_ref[0] = num / denom


def kernel(embedding, x):
    xb = x.reshape(_NBLK, _BLK, 256)
    out = pl.pallas_call(
        _cosine_block,
        grid=(_NBLK,),
        in_specs=[
            pl.BlockSpec((1, 256), lambda i: (0, 0)),
            pl.BlockSpec((1, _BLK, 256), lambda i: (i, 0, 0)),
        ],
        out_specs=pl.BlockSpec((1, 1, _BLK), lambda i: (i, 0, 0)),
        out_shape=jax.ShapeDtypeStruct((_NBLK, 1, _BLK), jnp.float32),
    )(embedding, xb)
    return out.reshape(100000)


# speedup vs baseline: 3.9555x; 3.0901x over previous
import jax
import jax.numpy as jnp
from jax.experimental import pallas as pl

_EPS = 1e-8
_NBLK = 20
_BLK = 5000


def _cosine_block(e_ref, x_ref, o_ref):
    x = x_ref[:]
    e = e_ref[:]
    dims = (((1,), (1,)), ((), ()))
    num = jax.lax.dot_general(e, x, dims,
                              preferred_element_type=jnp.float32)
    ones = jnp.ones((1, 256), jnp.float32)
    n2sq = jax.lax.dot_general(ones, x * x, dims,
                               preferred_element_type=jnp.float32)
    n2 = jnp.sqrt(n2sq)
    n1 = jnp.sqrt(jnp.sum(e * e))
    denom = jnp.maximum(n1, _EPS) * jnp.maximum(n2, _EPS)
    o_ref[0] = num / denom


def kernel(embedding, x):
    out = pl.pallas_call(
        _cosine_block,
        grid=(_NBLK,),
        in_specs=[
            pl.BlockSpec((1, 256), lambda i: (0, 0)),
            pl.BlockSpec((_BLK, 256), lambda i: (i, 0)),
        ],
        out_specs=pl.BlockSpec((1, 1, _BLK), lambda i: (i, 0, 0)),
        out_shape=jax.ShapeDtypeStruct((_NBLK, 1, _BLK), jnp.float32),
    )(embedding, x)
    return out.reshape(100000)


# speedup vs baseline: 4.4227x; 1.1181x over previous
import jax
import jax.numpy as jnp
from jax.experimental import pallas as pl

_EPS = 1e-8
_NBLK = 10
_BLK = 10000


def _cosine_block(e_ref, x_ref, o_ref):
    x = x_ref[:]
    e = e_ref[:]
    dims = (((1,), (1,)), ((), ()))
    num = jax.lax.dot_general(e, x, dims,
                              preferred_element_type=jnp.float32)
    ones = jnp.ones((1, 256), jnp.float32)
    n2sq = jax.lax.dot_general(ones, x * x, dims,
                               preferred_element_type=jnp.float32)
    n2 = jnp.sqrt(n2sq)
    n1 = jnp.sqrt(jnp.sum(e * e))
    denom = jnp.maximum(n1, _EPS) * jnp.maximum(n2, _EPS)
    o_ref[0] = num / denom


def kernel(embedding, x):
    out = pl.pallas_call(
        _cosine_block,
        grid=(_NBLK,),
        in_specs=[
            pl.BlockSpec((1, 256), lambda i: (0, 0)),
            pl.BlockSpec((_BLK, 256), lambda i: (i, 0)),
        ],
        out_specs=pl.BlockSpec((1, 1, _BLK), lambda i: (i, 0, 0)),
        out_shape=jax.ShapeDtypeStruct((_NBLK, 1, _BLK), jnp.float32),
    )(embedding, x)
    return out.reshape(100000)
